# baseline (speedup 1.0000x reference)
import functools

import numpy as np
import jax
import jax.numpy as jnp
from jax import lax
from jax.experimental import pallas as pl
from jax.experimental.pallas import tpu as pltpu

_EPS = 1e-5

_G1 = 21
_L1 = 441
_LX1 = 419
_L2 = 397
_L3 = 61


def _gelu(x):
    p = 0.3275911
    a1, a2, a3, a4, a5 = (0.254829592, -0.284496736, 1.421413741,
                          -1.453152027, 1.061405429)
    u = x * 0.7071067811865475
    au = jnp.abs(u)
    t = 1.0 / (1.0 + p * au)
    poly = ((((a5 * t + a4) * t + a3) * t + a2) * t + a1) * t
    e = 1.0 - poly * jnp.exp(-au * au)
    erf = jnp.where(u >= 0.0, e, -e)
    return 0.5 * x * (1.0 + erf)


def _msum(v, m):
    return jnp.sum(jnp.sum(v * m, axis=1, keepdims=True), axis=0,
                   keepdims=True)


def _pick_tn(n, max_tn):
    best = 1
    for d in range(1, min(max_tn, n) + 1):
        if n % d == 0 and (n // d) >= 2:
            best = d
    return best


def _head_kernel(xs_ref, w1_ref, b1_ref, g1_ref, bt1_ref, m1_ref,
                 w2_ref, b2_ref, g2_ref, bt2_ref, m2_ref, o_ref, *, tn):
    cnt1 = float(8 * 80 * 80)
    cnt2 = float(16 * 19 * 19)
    w1 = w1_ref[...]
    m1 = m1_ref[...]
    m2 = m2_ref[...]
    for i in range(tn):
        xs = xs_ref[i]
        x1 = jnp.concatenate(
            [xs[:, oi * _G1 + oj:oi * _G1 + oj + _LX1]
             for oi in (0, 1) for oj in (0, 1)], axis=0)
        acc = jnp.dot(w1, x1, preferred_element_type=jnp.float32)
        acc = jnp.concatenate(
            [acc, jnp.zeros((128, _L1 - _LX1), jnp.float32)], axis=1)
        acc = acc + b1_ref[...]
        s = _msum(acc, m1)
        mean = s * (1.0 / cnt1)
        cen = acc - mean
        v = _msum(cen * cen, m1)
        inv = lax.rsqrt(v * (1.0 / cnt1) + _EPS)
        z = _gelu(cen * inv * g1_ref[...] + bt1_ref[...])
        pe = []
        for p_ in (0, 1):
            for q_ in (0, 1):
                blk = None
                for di in (0, 1):
                    for dj in (0, 1):
                        r0 = ((2 * p_ + di) * 4 + (2 * q_ + dj)) * 8
                        piece = z[r0:r0 + 8]
                        blk = piece if blk is None else jnp.maximum(blk, piece)
                pe.append(blk)
        x2 = jnp.concatenate(
            [pe[(ki % 2) * 2 + (kj % 2)]
             [:, (ki == 2) * _G1 + (kj == 2):
                 (ki == 2) * _G1 + (kj == 2) + _L2]
             for ki in range(3) for kj in range(3)], axis=0)
        y2 = jnp.dot(w2_ref[...], x2, preferred_element_type=jnp.float32)
        y2 = y2 + b2_ref[...]
        s2 = _msum(y2, m2)
        mean2 = s2 * (1.0 / cnt2)
        cen2 = y2 - mean2
        v2 = _msum(cen2 * cen2, m2)
        inv2 = lax.rsqrt(v2 * (1.0 / cnt2) + _EPS)
        o_ref[i] = _gelu(cen2 * inv2 * g2_ref[...] + bt2_ref[...])


def _tail_kernel(z2_ref, p2_ref, w3_ref, b3_ref, g3_ref, bt3_ref, m3_ref,
                 q3_ref, w4_ref, b4_ref, g4_ref, bt4_ref, m4_ref,
                 o_ref, *, tn):
    cnt3 = float(32 * 7 * 7)
    cnt4 = float(64 * 3 * 3)
    p2 = p2_ref[...]
    q3 = q3_ref[...]
    m3 = m3_ref[...]
    for i in range(tn):
        z2 = z2_ref[i]
        pc = jnp.dot(z2, p2, preferred_element_type=jnp.float32)
        pooled2 = jnp.maximum(jnp.maximum(pc[:, 0:81], pc[:, 81:162]),
                              jnp.maximum(pc[:, 162:243], pc[:, 243:324]))
        x3 = jnp.concatenate(
            [pooled2[:, ki * 9 + kj:ki * 9 + kj + _L3]
             for ki in range(3) for kj in range(3)], axis=0)
        y3 = jnp.dot(w3_ref[...], x3, preferred_element_type=jnp.float32)
        y3 = y3 + b3_ref[...]
        s3 = _msum(y3, m3)
        mean3 = s3 * (1.0 / cnt3)
        cen3 = y3 - mean3
        v3 = _msum(cen3 * cen3, m3)
        inv3 = lax.rsqrt(v3 * (1.0 / cnt3) + _EPS)
        z3 = _gelu(cen3 * inv3 * g3_ref[...] + bt3_ref[...])
        qc = jnp.dot(z3, q3, preferred_element_type=jnp.float32)
        pooled3 = jnp.maximum(jnp.maximum(qc[:, 0:9], qc[:, 9:18]),
                              jnp.maximum(qc[:, 18:27], qc[:, 27:36]))
        pad = jnp.zeros((32, 4), jnp.float32)
        p3w = jnp.concatenate([pad, pooled3, pad], axis=1)
        x4 = jnp.concatenate(
            [p3w[:, ki * 3 + kj:9 + ki * 3 + kj]
             * m4_ref[ki * 3 + kj:ki * 3 + kj + 1, :]
             for ki in range(3) for kj in range(3)], axis=0)
        y4 = jnp.dot(w4_ref[...], x4, preferred_element_type=jnp.float32)
        y4 = y4 + b4_ref[...]
        s4 = jnp.sum(jnp.sum(y4, axis=1, keepdims=True), axis=0,
                     keepdims=True)
        mean4 = s4 * (1.0 / cnt4)
        cen4 = y4 - mean4
        v4 = jnp.sum(jnp.sum(cen4 * cen4, axis=1, keepdims=True), axis=0,
                     keepdims=True)
        inv4 = lax.rsqrt(v4 * (1.0 / cnt4) + _EPS)
        z4 = cen4 * inv4 * g4_ref[...] + bt4_ref[...]
        o_ref[i] = jnp.maximum(jnp.maximum(z4[:, 0:1], z4[:, 1:2]),
                               jnp.maximum(z4[:, 3:4], z4[:, 4:5]))


def _conv1_weight(w1):
    w1p = jnp.pad(w1.astype(jnp.float32), ((0, 0), (0, 0), (0, 1), (0, 1)))
    O = np.zeros((128, 256), np.int32)
    CI = np.zeros((128, 256), np.int32)
    KI = np.full((128, 256), 5, np.int32)
    KJ = np.full((128, 256), 5, np.int32)
    for i in range(4):
        for j in range(4):
            for o in range(8):
                r = (i * 4 + j) * 8 + o
                O[r, :] = o
                for oi in range(2):
                    for oj in range(2):
                        for p_ in range(4):
                            for q_ in range(4):
                                for ci in range(4):
                                    c = ((oi * 2 + oj) * 64
                                         + (p_ * 4 + q_) * 4 + ci)
                                    CI[r, c] = ci
                                    ki = 4 * oi + p_ - i
                                    kj = 4 * oj + q_ - j
                                    if 0 <= ki < 5 and 0 <= kj < 5:
                                        KI[r, c] = ki
                                        KJ[r, c] = kj
    return w1p[O, CI, KI, KJ]


def _affine1(a):
    a = a.astype(jnp.float32).reshape(8, 20, 4, 20, 4)
    a = a.transpose(2, 4, 0, 1, 3).reshape(128, 20, 20)
    return jnp.pad(a, ((0, 0), (0, 1), (0, 1))).reshape(128, _L1)


def _pool_sel(h, w, stride_grid, out_n):
    length = (h - 2) * stride_grid + (w - 1) + 1
    hp = out_n
    m = np.zeros((length, 4 * hp * hp), np.float32)
    for di in range(2):
        for dj in range(2):
            c0 = (di * 2 + dj) * hp * hp
            for r in range(hp):
                for s in range(hp):
                    m[(2 * r + di) * stride_grid + (2 * s + dj),
                      c0 + r * hp + s] = 1.0
    return jnp.asarray(m)


@functools.lru_cache(maxsize=None)
def _np_consts():
    lane1 = np.arange(_L1)
    m1 = (((lane1 // _G1) < 20) & ((lane1 % _G1) < 20)).astype(np.float32)
    lane2 = np.arange(_L2)
    m2 = ((lane2 % _G1) < 19).astype(np.float32)
    lane3 = np.arange(_L3)
    m3 = ((lane3 % 9) < 7).astype(np.float32)
    m4 = np.zeros((9, 9), np.float32)
    for ki in range(3):
        for kj in range(3):
            for r in range(3):
                for c in range(3):
                    if 0 <= r + ki - 1 < 3 and 0 <= c + kj - 1 < 3:
                        m4[ki * 3 + kj, r * 3 + c] = 1.0
    return (m1.reshape(1, _L1), m2.reshape(1, _L2), m3.reshape(1, _L3), m4)


def kernel(x, w1, b1, w2, b2, w3, b3, w4, b4,
           g1, be1, g2, be2, g3, be3, g4, be4):
    batch, horizon = x.shape[0], x.shape[1]
    n = batch * horizon
    xi = x.reshape(n, 4, 84, 84).astype(jnp.float32)

    xs = xi.reshape(n, 4, _G1, 4, _G1, 4)
    xs = xs.transpose(0, 3, 5, 1, 2, 4).reshape(n, 64, _L1)

    m1np, m2np, m3np, m4np = _np_consts()
    w1s = _conv1_weight(w1)
    b1c = jnp.tile(b1.astype(jnp.float32), 16).reshape(128, 1)
    g1s, bt1s = _affine1(g1), _affine1(be1)
    mask1 = jnp.asarray(m1np)

    w2s = w2.astype(jnp.float32).transpose(0, 2, 3, 1).reshape(16, 72)
    b2c = b2.astype(jnp.float32).reshape(16, 1)

    def _aff2(a):
        a = a.astype(jnp.float32)
        return jnp.pad(a, ((0, 0), (0, 0), (0, 2))).reshape(16, 399)[:, :_L2]

    g2s, bt2s = _aff2(g2), _aff2(be2)
    mask2 = jnp.asarray(m2np)

    tn_a = _pick_tn(n, 4)
    head = functools.partial(_head_kernel, tn=tn_a)
    z2 = pl.pallas_call(
        head,
        out_shape=jax.ShapeDtypeStruct((n, 16, _L2), jnp.float32),
        grid=(n // tn_a,),
        in_specs=[
            pl.BlockSpec((tn_a, 64, _L1), lambda g: (g, 0, 0)),
            pl.BlockSpec((128, 256), lambda g: (0, 0)),
            pl.BlockSpec((128, 1), lambda g: (0, 0)),
            pl.BlockSpec((128, _L1), lambda g: (0, 0)),
            pl.BlockSpec((128, _L1), lambda g: (0, 0)),
            pl.BlockSpec((1, _L1), lambda g: (0, 0)),
            pl.BlockSpec((16, 72), lambda g: (0, 0)),
            pl.BlockSpec((16, 1), lambda g: (0, 0)),
            pl.BlockSpec((16, _L2), lambda g: (0, 0)),
            pl.BlockSpec((16, _L2), lambda g: (0, 0)),
            pl.BlockSpec((1, _L2), lambda g: (0, 0)),
        ],
        out_specs=pl.BlockSpec((tn_a, 16, _L2), lambda g: (g, 0, 0)),
        compiler_params=pltpu.CompilerParams(
            dimension_semantics=("parallel",),
            vmem_limit_bytes=64 * 1024 * 1024),
    )(xs, w1s, b1c, g1s, bt1s, mask1, w2s, b2c, g2s, bt2s, mask2)

    p2 = _pool_sel(19, 19, _G1, 9)
    p2 = jnp.pad(p2, ((0, _L2 - p2.shape[0]), (0, 0)))
    q3 = _pool_sel(7, 7, 9, 3)
    q3 = jnp.pad(q3, ((0, _L3 - q3.shape[0]), (0, 0)))

    w3s = w3.astype(jnp.float32).transpose(0, 2, 3, 1).reshape(32, 144)
    b3c = b3.astype(jnp.float32).reshape(32, 1)
    g3s = jnp.pad(g3.astype(jnp.float32),
                  ((0, 0), (0, 0), (0, 2))).reshape(32, 63)[:, :_L3]
    bt3s = jnp.pad(be3.astype(jnp.float32),
                   ((0, 0), (0, 0), (0, 2))).reshape(32, 63)[:, :_L3]
    mask3 = jnp.asarray(m3np)
    mask4 = jnp.asarray(m4np)

    w4s = w4.astype(jnp.float32).transpose(0, 2, 3, 1).reshape(64, 288)
    b4c = b4.astype(jnp.float32).reshape(64, 1)
    g4s = g4.astype(jnp.float32).reshape(64, 9)
    bt4s = be4.astype(jnp.float32).reshape(64, 9)

    tn_b = _pick_tn(n, 8)
    tail = functools.partial(_tail_kernel, tn=tn_b)
    out = pl.pallas_call(
        tail,
        out_shape=jax.ShapeDtypeStruct((n, 64, 1), jnp.float32),
        grid=(n // tn_b,),
        in_specs=[
            pl.BlockSpec((tn_b, 16, _L2), lambda g: (g, 0, 0)),
            pl.BlockSpec((_L2, 324), lambda g: (0, 0)),
            pl.BlockSpec((32, 144), lambda g: (0, 0)),
            pl.BlockSpec((32, 1), lambda g: (0, 0)),
            pl.BlockSpec((32, _L3), lambda g: (0, 0)),
            pl.BlockSpec((32, _L3), lambda g: (0, 0)),
            pl.BlockSpec((1, _L3), lambda g: (0, 0)),
            pl.BlockSpec((_L3, 36), lambda g: (0, 0)),
            pl.BlockSpec((64, 288), lambda g: (0, 0)),
            pl.BlockSpec((64, 1), lambda g: (0, 0)),
            pl.BlockSpec((64, 9), lambda g: (0, 0)),
            pl.BlockSpec((64, 9), lambda g: (0, 0)),
            pl.BlockSpec((9, 9), lambda g: (0, 0)),
        ],
        out_specs=pl.BlockSpec((tn_b, 64, 1), lambda g: (g, 0, 0)),
        compiler_params=pltpu.CompilerParams(
            dimension_semantics=("parallel",),
            vmem_limit_bytes=64 * 1024 * 1024),
    )(z2, p2, w3s, b3c, g3s, bt3s, mask3, q3, w4s, b4c, g4s, bt4s, mask4)

    return out.reshape(batch, horizon, 64)


# speedup vs baseline: 4.2863x; 4.2863x over previous
import functools

import numpy as np
import jax
import jax.numpy as jnp
from jax import lax
from jax.experimental import pallas as pl
from jax.experimental.pallas import tpu as pltpu

_EPS = 1e-5

_LIN = 1764
_L1 = 1676
_L2 = 1585
_L3 = 61
_LP = 1497


def _gelu(x):
    p = 0.3275911
    a1, a2, a3, a4, a5 = (0.254829592, -0.284496736, 1.421413741,
                          -1.453152027, 1.061405429)
    u = x * 0.7071067811865475
    au = jnp.abs(u)
    t = 1.0 / (1.0 + p * au)
    poly = ((((a5 * t + a4) * t + a3) * t + a2) * t + a1) * t
    e = 1.0 - poly * jnp.exp(-au * au)
    erf = jnp.where(u >= 0.0, e, -e)
    return 0.5 * x * (1.0 + erf)


def _msum(v, m):
    return jnp.sum(jnp.sum(v * m, axis=1, keepdims=True), axis=0,
                   keepdims=True)


def _sum2(v):
    return jnp.sum(jnp.sum(v, axis=1, keepdims=True), axis=0, keepdims=True)


def _pick_tn(n, max_tn):
    best = 1
    for d in range(1, min(max_tn, n) + 1):
        if n % d == 0 and (n // d) >= 2:
            best = d
    return best


def _fused_kernel(xs_ref, w1_ref, b1_ref, g1_ref, bt1_ref, m1_ref,
                  w2_ref, b2_ref, g2_ref, bt2_ref, m2_ref, c2_ref,
                  w3_ref, b3_ref, g3_ref, bt3_ref, m3_ref, q3_ref,
                  w4_ref, b4_ref, g4_ref, bt4_ref, m4_ref,
                  o_ref, *, tn):
    cnt1 = float(8 * 80 * 80)
    cnt2 = float(16 * 19 * 19)
    cnt3 = float(32 * 7 * 7)
    cnt4 = float(64 * 3 * 3)
    w1 = w1_ref[...]
    m1 = m1_ref[...]
    m2 = m2_ref[...]
    for i in range(tn):
        xs = xs_ref[i]
        x1 = jnp.concatenate(
            [xs[:, oi * 84 + kj:oi * 84 + kj + _L1]
             for oi in (0, 1) for kj in range(5)], axis=0)
        acc = jnp.dot(w1, x1, preferred_element_type=jnp.float32)
        acc = acc + b1_ref[...]
        s = _msum(acc, m1)
        mean = s * (1.0 / cnt1)
        cen = acc - mean
        v = _msum(cen * cen, m1)
        inv = lax.rsqrt(v * (1.0 / cnt1) + _EPS)
        z1 = _gelu(cen * inv * g1_ref[...] + bt1_ref[...])
        pe0 = jnp.maximum(z1[0:8], z1[8:16])
        pe1 = jnp.maximum(z1[16:24], z1[24:32])
        p0 = jnp.maximum(pe0[:, 0:_L1 - 1], pe0[:, 1:_L1])
        p1 = jnp.maximum(pe1[:, 0:_L1 - 1], pe1[:, 1:_L1])
        pool1 = (p0, p1)
        x2 = jnp.concatenate(
            [pool1[ki % 2][:, (ki == 2) * 84 + 2 * kj:
                           (ki == 2) * 84 + 2 * kj + _L2]
             for ki in range(3) for kj in range(3)], axis=0)
        y2 = jnp.dot(w2_ref[...], x2, preferred_element_type=jnp.float32)
        y2 = y2 + b2_ref[...]
        s2 = _msum(y2, m2)
        mean2 = s2 * (1.0 / cnt2)
        cen2 = y2 - mean2
        v2 = _msum(cen2 * cen2, m2)
        inv2 = lax.rsqrt(v2 * (1.0 / cnt2) + _EPS)
        z2 = _gelu(cen2 * inv2 * g2_ref[...] + bt2_ref[...])
        pm = jnp.maximum(z2[:, 0:_L2 - 84], z2[:, 84:_L2])
        pm = jnp.maximum(pm[:, 0:_LP], pm[:, 4:_LP + 4])
        pooled2 = jnp.dot(pm, c2_ref[...],
                          preferred_element_type=jnp.float32)
        x3 = jnp.concatenate(
            [pooled2[:, ki * 9 + kj:ki * 9 + kj + _L3]
             for ki in range(3) for kj in range(3)], axis=0)
        y3 = jnp.dot(w3_ref[...], x3, preferred_element_type=jnp.float32)
        y3 = y3 + b3_ref[...]
        s3 = _msum(y3, m3_ref[...])
        mean3 = s3 * (1.0 / cnt3)
        cen3 = y3 - mean3
        v3 = _msum(cen3 * cen3, m3_ref[...])
        inv3 = lax.rsqrt(v3 * (1.0 / cnt3) + _EPS)
        z3 = _gelu(cen3 * inv3 * g3_ref[...] + bt3_ref[...])
        qc = jnp.dot(z3, q3_ref[...],
                     preferred_element_type=jnp.float32)
        pooled3 = jnp.maximum(jnp.maximum(qc[:, 0:9], qc[:, 9:18]),
                              jnp.maximum(qc[:, 18:27], qc[:, 27:36]))
        pad = jnp.zeros((32, 4), jnp.float32)
        p3w = jnp.concatenate([pad, pooled3, pad], axis=1)
        x4 = jnp.concatenate(
            [p3w[:, ki * 3 + kj:9 + ki * 3 + kj]
             * m4_ref[ki * 3 + kj:ki * 3 + kj + 1, :]
             for ki in range(3) for kj in range(3)], axis=0)
        y4 = jnp.dot(w4_ref[...], x4, preferred_element_type=jnp.float32)
        y4 = y4 + b4_ref[...]
        s4 = _sum2(y4)
        mean4 = s4 * (1.0 / cnt4)
        cen4 = y4 - mean4
        v4 = _sum2(cen4 * cen4)
        inv4 = lax.rsqrt(v4 * (1.0 / cnt4) + _EPS)
        z4 = cen4 * inv4 * g4_ref[...] + bt4_ref[...]
        o_ref[i] = jnp.maximum(jnp.maximum(z4[:, 0:1], z4[:, 1:2]),
                               jnp.maximum(z4[:, 3:4], z4[:, 4:5]))


def _conv1_weight(w1):
    w1p = jnp.pad(w1.astype(jnp.float32), ((0, 0), (0, 0), (0, 1), (0, 0)))
    O = np.zeros((32, 160), np.int32)
    CI = np.zeros((32, 160), np.int32)
    KI = np.full((32, 160), 5, np.int32)
    KJ = np.zeros((32, 160), np.int32)
    for i in range(4):
        for o in range(8):
            r = i * 8 + o
            O[r, :] = o
            for oi in range(2):
                for kj in range(5):
                    for ci in range(4):
                        for p_ in range(4):
                            c = (oi * 5 + kj) * 16 + ci * 4 + p_
                            CI[r, c] = ci
                            KJ[r, c] = kj
                            ki = 4 * oi + p_ - i
                            if 0 <= ki < 5:
                                KI[r, c] = ki
    return w1p[O, CI, KI, KJ]


@functools.lru_cache(maxsize=None)
def _np_consts():
    lane1 = np.arange(_L1)
    m1 = ((lane1 % 84) < 80).astype(np.float32).reshape(1, _L1)
    lane2 = np.arange(_L2)
    c = lane2 % 84
    m2 = ((c % 4 == 0) & (c <= 72)).astype(np.float32).reshape(1, _L2)
    lane3 = np.arange(_L3)
    m3 = ((lane3 % 9) < 7).astype(np.float32).reshape(1, _L3)
    c2 = np.zeros((_LP, 81), np.float32)
    for r in range(9):
        for s_ in range(9):
            c2[168 * r + 8 * s_, r * 9 + s_] = 1.0
    q3 = np.zeros((_L3, 36), np.float32)
    for di in range(2):
        for dj in range(2):
            for r in range(3):
                for s_ in range(3):
                    q3[(2 * r + di) * 9 + (2 * s_ + dj),
                       (di * 2 + dj) * 9 + r * 3 + s_] = 1.0
    m4 = np.zeros((9, 9), np.float32)
    for ki in range(3):
        for kj in range(3):
            for r in range(3):
                for cc in range(3):
                    if 0 <= r + ki - 1 < 3 and 0 <= cc + kj - 1 < 3:
                        m4[ki * 3 + kj, r * 3 + cc] = 1.0
    return m1, m2, m3, c2, q3, m4


def _affine1(a):
    a = a.astype(jnp.float32).reshape(8, 20, 4, 80)
    a = a.transpose(2, 0, 1, 3)
    a = jnp.pad(a, ((0, 0), (0, 0), (0, 0), (0, 4)))
    return a.reshape(32, 1680)[:, :_L1]


def _affine2(a):
    a = a.astype(jnp.float32)[:, :, :, None]
    a = jnp.pad(a, ((0, 0), (0, 0), (0, 0), (0, 3)))
    a = jnp.pad(a, ((0, 0), (0, 0), (0, 2), (0, 0)))
    return a.reshape(16, 1596)[:, :_L2]


def _affine3(a):
    a = jnp.pad(a.astype(jnp.float32), ((0, 0), (0, 0), (0, 2)))
    return a.reshape(32, 63)[:, :_L3]


def kernel(x, w1, b1, w2, b2, w3, b3, w4, b4,
           g1, be1, g2, be2, g3, be3, g4, be4):
    batch, horizon = x.shape[0], x.shape[1]
    n = batch * horizon
    xi = x.reshape(n, 4, 84, 84).astype(jnp.float32)

    xs = xi.reshape(n, 4, 21, 4, 84).transpose(0, 1, 3, 2, 4)
    xs = xs.reshape(n, 16, _LIN)

    m1np, m2np, m3np, c2np, q3np, m4np = _np_consts()
    w1s = _conv1_weight(w1)
    b1c = jnp.tile(b1.astype(jnp.float32), 4).reshape(32, 1)
    g1s, bt1s = _affine1(g1), _affine1(be1)

    w2s = w2.astype(jnp.float32).transpose(0, 2, 3, 1).reshape(16, 72)
    b2c = b2.astype(jnp.float32).reshape(16, 1)
    g2s, bt2s = _affine2(g2), _affine2(be2)

    w3s = w3.astype(jnp.float32).transpose(0, 2, 3, 1).reshape(32, 144)
    b3c = b3.astype(jnp.float32).reshape(32, 1)
    g3s, bt3s = _affine3(g3), _affine3(be3)

    w4s = w4.astype(jnp.float32).transpose(0, 2, 3, 1).reshape(64, 288)
    b4c = b4.astype(jnp.float32).reshape(64, 1)
    g4s = g4.astype(jnp.float32).reshape(64, 9)
    bt4s = be4.astype(jnp.float32).reshape(64, 9)

    tn = _pick_tn(n, 4)
    kern = functools.partial(_fused_kernel, tn=tn)

    def _const(shape):
        return pl.BlockSpec(shape, lambda g, _nd=len(shape): (0,) * _nd)

    out = pl.pallas_call(
        kern,
        out_shape=jax.ShapeDtypeStruct((n, 64, 1), jnp.float32),
        grid=(n // tn,),
        in_specs=[
            pl.BlockSpec((tn, 16, _LIN), lambda g: (g, 0, 0)),
            _const((32, 160)), _const((32, 1)),
            _const((32, _L1)), _const((32, _L1)), _const((1, _L1)),
            _const((16, 72)), _const((16, 1)),
            _const((16, _L2)), _const((16, _L2)), _const((1, _L2)),
            _const((_LP, 81)),
            _const((32, 144)), _const((32, 1)),
            _const((32, _L3)), _const((32, _L3)), _const((1, _L3)),
            _const((_L3, 36)),
            _const((64, 288)), _const((64, 1)),
            _const((64, 9)), _const((64, 9)), _const((9, 9)),
        ],
        out_specs=pl.BlockSpec((tn, 64, 1), lambda g: (g, 0, 0)),
        compiler_params=pltpu.CompilerParams(
            dimension_semantics=("parallel",),
            vmem_limit_bytes=64 * 1024 * 1024),
    )(xs, w1s, b1c, g1s, bt1s, jnp.asarray(m1np),
      w2s, b2c, g2s, bt2s, jnp.asarray(m2np), jnp.asarray(c2np),
      w3s, b3c, g3s, bt3s, jnp.asarray(m3np), jnp.asarray(q3np),
      w4s, b4c, g4s, bt4s, jnp.asarray(m4np))

    return out.reshape(batch, horizon, 64)
